# baseline (speedup 1.0000x reference)
import functools

import jax
import jax.numpy as jnp
import numpy as np
from jax import lax
from jax.experimental import pallas as pl
from jax.experimental.pallas import tpu as pltpu
from jax.experimental.pallas import tpu_sc as plsc

_N_BINS = 20
_N = 1_000_000
_C = 100

_NW = 32
_RW = 31248
_BASE_EXTRA = _NW * _RW
_CHUNK = 496
_NCHUNK = _RW // _CHUNK
_GROUPS = _CHUNK // 16
_TBL = 16 * (_N_BINS + 1)

_BOUNDS = [float(v) for v in np.arange(_N_BINS + 1, dtype=np.float32)
           * np.float32(1.0 / _N_BINS)]


_NCH = 4


def _process_group(xbuf, lblbuf, cnt_t, conf_t, acc_t, g):
    lanes = lax.iota(jnp.int32, 16)
    rows = g * 16 + lanes

    def col_step(j, carry):
        out = []
        for k in range(_NCH):
            m, am = carry[k]
            c = _NCH * j + k
            v = plsc.load_gather(xbuf, [rows, jnp.full((16,), c, jnp.int32)])
            gt = v > m
            am = jnp.where(gt, c, am)
            m = jnp.maximum(m, v)
            out.append((m, am))
        return tuple(out)

    init = tuple(
        (plsc.load_gather(xbuf, [rows, jnp.full((16,), k, jnp.int32)]),
         jnp.full((16,), k, jnp.int32))
        for k in range(_NCH)
    )
    chains = lax.fori_loop(1, _C // _NCH, col_step, init, unroll=2)

    def merge(a, b):
        ma, aa = a
        mb, ab = b
        gt = mb > ma
        eq = mb == ma
        am = jnp.where(gt, ab, jnp.where(eq, jnp.minimum(aa, ab), aa))
        return (jnp.maximum(ma, mb), am)

    m, am = merge(merge(chains[0], chains[1]), merge(chains[2], chains[3]))

    lbl = lblbuf[pl.ds(g * 16, 16)]
    acc = jnp.where(am == lbl, 1.0, 0.0)

    t = (m * np.float32(_N_BINS)).astype(jnp.int32)
    bt = t.astype(jnp.float32) * np.float32(1.0 / _N_BINS)
    k = t + jnp.where(m <= bt, 0, 1)
    idx = k * 16 + lanes
    plsc.addupdate_scatter(cnt_t, [idx], jnp.ones((16,), jnp.float32))
    plsc.addupdate_scatter(conf_t, [idx], m)
    plsc.addupdate_scatter(acc_t, [idx], acc)


def _sc_partials(logits, labels):
    mesh = plsc.VectorSubcoreMesh(core_axis_name="c", subcore_axis_name="s")

    @functools.partial(
        pl.kernel,
        out_type=jax.ShapeDtypeStruct((_NW * 3 * _TBL,), jnp.float32),
        mesh=mesh,
        scratch_types=[
            pltpu.VMEM((_CHUNK, _C), jnp.float32),
            pltpu.VMEM((_CHUNK, _C), jnp.float32),
            pltpu.VMEM((_CHUNK,), jnp.int32),
            pltpu.VMEM((_CHUNK,), jnp.int32),
            pltpu.VMEM((_TBL,), jnp.float32),
            pltpu.VMEM((_TBL,), jnp.float32),
            pltpu.VMEM((_TBL,), jnp.float32),
            pltpu.SemaphoreType.DMA,
            pltpu.SemaphoreType.DMA,
            pltpu.SemaphoreType.DMA,
            pltpu.SemaphoreType.DMA,
        ],
        compiler_params=pltpu.CompilerParams(needs_layout_passes=False),
    )
    def sc_kernel(logits_hbm, labels_hbm, out_hbm,
                  xb0, xb1, lb0, lb1, cnt_t, conf_t, acc_t,
                  sx0, sx1, sl0, sl1):
        wid = lax.axis_index("s") * 2 + lax.axis_index("c")
        base = wid * _RW

        zero16 = jnp.zeros((16,), jnp.float32)
        for i in range(_TBL // 16):
            cnt_t[pl.ds(16 * i, 16)] = zero16
            conf_t[pl.ds(16 * i, 16)] = zero16
            acc_t[pl.ds(16 * i, 16)] = zero16

        def start(c, xb, lb, sx, sl):
            r0 = base + c * _CHUNK
            pltpu.make_async_copy(
                logits_hbm.at[pl.ds(r0, _CHUNK), :], xb, sx).start()
            pltpu.make_async_copy(
                labels_hbm.at[pl.ds(r0, _CHUNK)], lb, sl).start()

        def wait(xb, lb, sx, sl):
            pltpu.make_async_copy(
                logits_hbm.at[pl.ds(0, _CHUNK), :], xb, sx).wait()
            pltpu.make_async_copy(
                labels_hbm.at[pl.ds(0, _CHUNK)], lb, sl).wait()

        def process(xb, lb):
            for g in range(_GROUPS):
                _process_group(xb, lb, cnt_t, conf_t, acc_t, g)

        start(0, xb0, lb0, sx0, sl0)
        start(1, xb1, lb1, sx1, sl1)

        def chunk_pair(t, carry):
            wait(xb0, lb0, sx0, sl0)
            process(xb0, lb0)

            @pl.when(2 * t + 2 < _NCHUNK)
            def _start0():
                start(2 * t + 2, xb0, lb0, sx0, sl0)

            wait(xb1, lb1, sx1, sl1)
            process(xb1, lb1)

            @pl.when(2 * t + 3 < _NCHUNK)
            def _start1():
                start(2 * t + 3, xb1, lb1, sx1, sl1)

            return carry

        lax.fori_loop(0, _NCHUNK // 2, chunk_pair, 0)
        wait(xb0, lb0, sx0, sl0)
        process(xb0, lb0)

        @pl.when(wid < 4)
        def _extra():
            r0 = _BASE_EXTRA + wid * 16
            pltpu.make_async_copy(
                logits_hbm.at[pl.ds(r0, 16), :],
                xb1.at[pl.ds(0, 16), :], sx1).start()
            pltpu.make_async_copy(
                labels_hbm.at[pl.ds(r0, 16)], lb1.at[pl.ds(0, 16)], sl1).start()
            pltpu.make_async_copy(
                logits_hbm.at[pl.ds(0, 16), :],
                xb1.at[pl.ds(0, 16), :], sx1).wait()
            pltpu.make_async_copy(
                labels_hbm.at[pl.ds(0, 16)], lb1.at[pl.ds(0, 16)], sl1).wait()
            _process_group(xb1, lb1, cnt_t, conf_t, acc_t, 0)

        obase = wid * 3 * _TBL
        pltpu.sync_copy(cnt_t, out_hbm.at[pl.ds(obase, _TBL)])
        pltpu.sync_copy(conf_t, out_hbm.at[pl.ds(obase + _TBL, _TBL)])
        pltpu.sync_copy(acc_t, out_hbm.at[pl.ds(obase + 2 * _TBL, _TBL)])

    return sc_kernel(logits, labels)


def _finish_body(p_ref, ece_ref, ys_ref):
    p = p_ref[...]
    t = jnp.sum(p.reshape(_NW, 3, _N_BINS + 1, 16), axis=(0, 3))
    cnt = t[0, 1:]
    sconf = t[1, 1:]
    sacc = t[2, 1:]
    has = cnt > 0.0
    denom = jnp.maximum(cnt, 1.0)
    acc_in = jnp.where(has, sacc / denom, 0.0)
    conf_in = jnp.where(has, sconf / denom, 0.0)
    prop = cnt * (1.0 / _N)
    ece = jnp.sum(jnp.where(has, jnp.abs(conf_in - acc_in) * prop, 0.0))
    ece_ref[...] = ece.reshape(1, 1)
    ys_ref[...] = acc_in.reshape(1, _N_BINS)


def kernel(logits, labels):
    partials = _sc_partials(logits, labels).reshape(_NW, 3, _TBL)
    ece2, ys2 = pl.pallas_call(
        _finish_body,
        out_shape=[
            jax.ShapeDtypeStruct((1, 1), jnp.float32),
            jax.ShapeDtypeStruct((1, _N_BINS), jnp.float32),
        ],
    )(partials)
    return (ece2.reshape(1), ys2.reshape(_N_BINS))


# speedup vs baseline: 2.1057x; 2.1057x over previous
import functools

import jax
import jax.numpy as jnp
import numpy as np
from jax import lax
from jax.experimental import pallas as pl
from jax.experimental.pallas import tpu as pltpu
from jax.experimental.pallas import tpu_sc as plsc

_N_BINS = 20
_N = 1_000_000
_C = 100

_NW = 32
_RW = 31248
_BASE_EXTRA = _NW * _RW
_CHUNK = 496
_NCHUNK = _RW // _CHUNK
_GROUPS = _CHUNK // 16
_TBL = 16 * (_N_BINS + 1)

_BOUNDS = [float(v) for v in np.arange(_N_BINS + 1, dtype=np.float32)
           * np.float32(1.0 / _N_BINS)]


_NCH = 4
_CB = _C // _NCH


def _process_group(xbuf, lblbuf, cnt_t, conf_t, acc_t, g):
    lanes = lax.iota(jnp.int32, 16)
    rows = g * 16 + lanes
    himask = jnp.full((16,), ~jnp.int32(127))

    def col_step(j, carry):
        w = carry[0]
        keys = list(carry[1])
        vals = list(carry[2])
        for k in range(_NCH):
            cvec = w + (_CB * k)
            v = plsc.load_gather(xbuf, [rows, cvec])
            b = lax.bitcast_convert_type(v, jnp.int32)
            key = (b & himask) | (127 - cvec)
            keys[k] = jnp.maximum(keys[k], key)
            vals[k] = jnp.maximum(vals[k], v)
        w = w + 1
        w = jnp.where(w == _CB, 0, w)
        return (w, tuple(keys), tuple(vals))

    zero = jnp.zeros((16,), jnp.int32)
    init = (lanes % _CB,
            (zero,) * _NCH,
            (jnp.full((16,), -1.0),) * _NCH)
    _, keys, vals = lax.fori_loop(0, _CB, col_step, init, unroll=2)

    m = jnp.maximum(jnp.maximum(vals[0], vals[1]),
                    jnp.maximum(vals[2], vals[3]))
    kf = jnp.maximum(jnp.maximum(keys[0], keys[1]),
                     jnp.maximum(keys[2], keys[3]))
    am = 127 - (kf & 127)

    lbl = lblbuf[pl.ds(g * 16, 16)]
    acc = jnp.where(am == lbl, 1.0, 0.0)

    t = (m * np.float32(_N_BINS)).astype(jnp.int32)
    bt = t.astype(jnp.float32) * np.float32(1.0 / _N_BINS)
    k = t + jnp.where(m <= bt, 0, 1)
    idx = k * 16 + lanes
    plsc.addupdate_scatter(cnt_t, [idx], jnp.ones((16,), jnp.float32))
    plsc.addupdate_scatter(conf_t, [idx], m)
    plsc.addupdate_scatter(acc_t, [idx], acc)


def _sc_partials(logits, labels):
    mesh = plsc.VectorSubcoreMesh(core_axis_name="c", subcore_axis_name="s")

    @functools.partial(
        pl.kernel,
        out_type=jax.ShapeDtypeStruct((_NW * 3 * _TBL,), jnp.float32),
        mesh=mesh,
        scratch_types=[
            pltpu.VMEM((_CHUNK, _C), jnp.float32),
            pltpu.VMEM((_CHUNK, _C), jnp.float32),
            pltpu.VMEM((_CHUNK,), jnp.int32),
            pltpu.VMEM((_CHUNK,), jnp.int32),
            pltpu.VMEM((_TBL,), jnp.float32),
            pltpu.VMEM((_TBL,), jnp.float32),
            pltpu.VMEM((_TBL,), jnp.float32),
            pltpu.SemaphoreType.DMA,
            pltpu.SemaphoreType.DMA,
            pltpu.SemaphoreType.DMA,
            pltpu.SemaphoreType.DMA,
        ],
        compiler_params=pltpu.CompilerParams(needs_layout_passes=False),
    )
    def sc_kernel(logits_hbm, labels_hbm, out_hbm,
                  xb0, xb1, lb0, lb1, cnt_t, conf_t, acc_t,
                  sx0, sx1, sl0, sl1):
        wid = lax.axis_index("s") * 2 + lax.axis_index("c")
        base = wid * _RW

        zero16 = jnp.zeros((16,), jnp.float32)
        for i in range(_TBL // 16):
            cnt_t[pl.ds(16 * i, 16)] = zero16
            conf_t[pl.ds(16 * i, 16)] = zero16
            acc_t[pl.ds(16 * i, 16)] = zero16

        def start(c, xb, lb, sx, sl):
            r0 = base + c * _CHUNK
            pltpu.make_async_copy(
                logits_hbm.at[pl.ds(r0, _CHUNK), :], xb, sx).start()
            pltpu.make_async_copy(
                labels_hbm.at[pl.ds(r0, _CHUNK)], lb, sl).start()

        def wait(xb, lb, sx, sl):
            pltpu.make_async_copy(
                logits_hbm.at[pl.ds(0, _CHUNK), :], xb, sx).wait()
            pltpu.make_async_copy(
                labels_hbm.at[pl.ds(0, _CHUNK)], lb, sl).wait()

        def process(xb, lb):
            for g in range(_GROUPS):
                _process_group(xb, lb, cnt_t, conf_t, acc_t, g)

        start(0, xb0, lb0, sx0, sl0)
        start(1, xb1, lb1, sx1, sl1)

        def chunk_pair(t, carry):
            wait(xb0, lb0, sx0, sl0)
            process(xb0, lb0)

            @pl.when(2 * t + 2 < _NCHUNK)
            def _start0():
                start(2 * t + 2, xb0, lb0, sx0, sl0)

            wait(xb1, lb1, sx1, sl1)
            process(xb1, lb1)

            @pl.when(2 * t + 3 < _NCHUNK)
            def _start1():
                start(2 * t + 3, xb1, lb1, sx1, sl1)

            return carry

        lax.fori_loop(0, _NCHUNK // 2, chunk_pair, 0)
        wait(xb0, lb0, sx0, sl0)
        process(xb0, lb0)

        @pl.when(wid < 4)
        def _extra():
            r0 = _BASE_EXTRA + wid * 16
            pltpu.make_async_copy(
                logits_hbm.at[pl.ds(r0, 16), :],
                xb1.at[pl.ds(0, 16), :], sx1).start()
            pltpu.make_async_copy(
                labels_hbm.at[pl.ds(r0, 16)], lb1.at[pl.ds(0, 16)], sl1).start()
            pltpu.make_async_copy(
                logits_hbm.at[pl.ds(0, 16), :],
                xb1.at[pl.ds(0, 16), :], sx1).wait()
            pltpu.make_async_copy(
                labels_hbm.at[pl.ds(0, 16)], lb1.at[pl.ds(0, 16)], sl1).wait()
            _process_group(xb1, lb1, cnt_t, conf_t, acc_t, 0)

        obase = wid * 3 * _TBL
        pltpu.sync_copy(cnt_t, out_hbm.at[pl.ds(obase, _TBL)])
        pltpu.sync_copy(conf_t, out_hbm.at[pl.ds(obase + _TBL, _TBL)])
        pltpu.sync_copy(acc_t, out_hbm.at[pl.ds(obase + 2 * _TBL, _TBL)])

    return sc_kernel(logits, labels)


def _finish_body(p_ref, ece_ref, ys_ref):
    p = p_ref[...]
    t = jnp.sum(p.reshape(_NW, 3, _N_BINS + 1, 16), axis=(0, 3))
    cnt = t[0, 1:]
    sconf = t[1, 1:]
    sacc = t[2, 1:]
    has = cnt > 0.0
    denom = jnp.maximum(cnt, 1.0)
    acc_in = jnp.where(has, sacc / denom, 0.0)
    conf_in = jnp.where(has, sconf / denom, 0.0)
    prop = cnt * (1.0 / _N)
    ece = jnp.sum(jnp.where(has, jnp.abs(conf_in - acc_in) * prop, 0.0))
    ece_ref[...] = ece.reshape(1, 1)
    ys_ref[...] = acc_in.reshape(1, _N_BINS)


def kernel(logits, labels):
    partials = _sc_partials(logits, labels).reshape(_NW, 3, _TBL)
    ece2, ys2 = pl.pallas_call(
        _finish_body,
        out_shape=[
            jax.ShapeDtypeStruct((1, 1), jnp.float32),
            jax.ShapeDtypeStruct((1, _N_BINS), jnp.float32),
        ],
    )(partials)
    return (ece2.reshape(1), ys2.reshape(_N_BINS))
